# baseline (speedup 1.0000x reference)
import functools

import jax
import jax.numpy as jnp
from jax import lax
from jax.experimental import pallas as pl
from jax.experimental.pallas import tpu as pltpu
from jax.experimental.pallas import tpu_sc as plsc

N = 10000
NPAD = 10240
D = 256
H1 = 4
F = 256
E = 160000
EP = E + N
BLK = 1024
CN = 160
NCHUNK = NPAD // CN
EPAD = 172032
NEG = 0.2
BETA = 0.5
CC = 1.0


def _tc_body(x_ref, w_ref, a_ref, h_ref, as_ref, ad_ref):
    h = jnp.dot(x_ref[...], w_ref[...], preferred_element_type=jnp.float32)
    h_ref[...] = h
    sd = jnp.dot(h, a_ref[...], preferred_element_type=jnp.float32)
    as_ref[...] = sd[:, :4]
    ad_ref[...] = sd[:, 4:]


def _tc_layer(xp, W, A):
    K, M = W.shape
    R = 2048
    f = pl.pallas_call(
        _tc_body,
        grid=(NPAD // R,),
        in_specs=[
            pl.BlockSpec((R, K), lambda i: (i, 0)),
            pl.BlockSpec((K, M), lambda i: (0, 0)),
            pl.BlockSpec((M, 8), lambda i: (0, 0)),
        ],
        out_specs=[
            pl.BlockSpec((R, M), lambda i: (i, 0)),
            pl.BlockSpec((R, 4), lambda i: (i, 0)),
            pl.BlockSpec((R, 4), lambda i: (i, 0)),
        ],
        out_shape=[
            jax.ShapeDtypeStruct((NPAD, M), jnp.float32),
            jax.ShapeDtypeStruct((NPAD, 4), jnp.float32),
            jax.ShapeDtypeStruct((NPAD, 4), jnp.float32),
        ],
    )
    return f(xp, W, A)


def _ext(sev, pos):
    return sev[pl.ds(pos, 16)][0]


def _sc_edge_body(H, h_hbm, as_hbm, ad_hbm, src_hbm, dst_hbm, se_hbm, b_hbm,
                  out_hbm,
                  asv, adc, sv, acc, srcb, dstb, rowbuf, rowbuf2, sev, bv,
                  dbuf, mbuf, abuf, sem, sem2):
    cid = lax.axis_index("c")
    sid = lax.axis_index("s")
    wid = sid * 2 + cid
    io = lax.iota(jnp.int32, 16)
    pltpu.sync_copy(as_hbm, asv)
    pltpu.sync_copy(se_hbm, sev.at[pl.ds(0, 128)])
    pltpu.sync_copy(b_hbm, bv)

    def _edge_w(srcv, doff):
        ws = []
        for h in range(H):
            a_s = plsc.load_gather(asv, [srcv * 4 + h])
            a_d = plsc.load_gather(adc, [doff * 4 + h])
            ev = a_s + a_d
            ev = jnp.where(ev >= 0.0, ev, NEG * ev)
            ws.append(jnp.exp(ev))
        return ws

    for k in range(NCHUNK // 32):
        c = wid * (NCHUNK // 32) + k
        base = c * CN
        start = _ext(sev, c)
        end = _ext(sev, c + NCHUNK)
        e0 = (start // 8) * 8
        nblk = (end - e0 + (BLK - 1)) // BLK
        pltpu.sync_copy(ad_hbm.at[pl.ds(base * 4, CN * 4)], adc)

        zf = jnp.zeros((16,), jnp.float32)

        def zs(i, _):
            sv[pl.ds(i * 16, 16)] = zf
            return 0
        lax.fori_loop(0, CN * H // 16, zs, 0)

        def za(i, _):
            acc[pl.ds(i * 16, 16)] = zf
            return 0
        lax.fori_loop(0, CN * F // 16, za, 0)

        def pass1_blk(b, _):
            eb = e0 + b * BLK
            pltpu.sync_copy(src_hbm.at[pl.ds(eb, BLK)], srcb)
            pltpu.sync_copy(dst_hbm.at[pl.ds(eb, BLK)], dstb)

            def grp(g, _):
                gb = g * 16
                srcv = srcb[pl.ds(gb, 16)]
                dstv = dstb[pl.ds(gb, 16)]
                gidx = eb + gb + io
                m = (gidx >= start) & (gidx < end)
                doff = jnp.clip(dstv - base, 0, CN - 1)
                ws = _edge_w(srcv, doff)
                for h in range(H):
                    plsc.addupdate_scatter(sv, [doff * H + h], ws[h], mask=m)
                return 0
            lax.fori_loop(0, BLK // 16, grp, 0)
            return 0
        lax.fori_loop(0, nblk, pass1_blk, 0)

        def inv(i, _):
            s16 = sv[pl.ds(i * 16, 16)]
            sv[pl.ds(i * 16, 16)] = 1.0 / (s16 + 1e-16)
            return 0
        lax.fori_loop(0, CN * H // 16, inv, 0)

        def pass2_blk(b, _):
            eb = e0 + b * BLK
            pltpu.sync_copy(src_hbm.at[pl.ds(eb, BLK)], srcb)
            pltpu.sync_copy(dst_hbm.at[pl.ds(eb, BLK)], dstb)

            def issue(g, buf, sm):
                srcv = srcb[pl.ds(g * 16, 16)]
                pltpu.async_copy(h_hbm.at[srcv], buf, sm)

            def waitbuf(buf, sm):
                pltpu.make_async_copy(h_hbm.at[pl.ds(0, 16)], buf, sm).wait()

            def process(g, buf):
                gb = g * 16
                srcv = srcb[pl.ds(gb, 16)]
                dstv = dstb[pl.ds(gb, 16)]
                gidx = eb + gb + io
                m = (gidx >= start) & (gidx < end)
                doff = jnp.clip(dstv - base, 0, CN - 1)
                ws = _edge_w(srcv, doff)
                for h in range(H):
                    invv = plsc.load_gather(sv, [doff * H + h])
                    abuf[pl.ds(h * 16, 16)] = ws[h] * invv
                dbuf[pl.ds(0, 16)] = doff
                mbuf[pl.ds(0, 16)] = m.astype(jnp.int32)

                def edge(l, _):
                    ml = mbuf[pl.ds(l, 16)][0] > 0

                    @pl.when(ml)
                    def _():
                        dof = dbuf[pl.ds(l, 16)][0]
                        als = [abuf[pl.ds(h * 16 + l, 16)][0]
                               for h in range(H)]
                        for j in range(F // 16):
                            v = None
                            for h in range(H):
                                r = buf[l, pl.ds(h * F + j * 16, 16)]
                                v = als[h] * r if v is None else v + als[h] * r
                            plsc.addupdate(acc.at[pl.ds(dof * F + j * 16, 16)],
                                           v)
                    return 0
                lax.fori_loop(0, 16, edge, 0)

            issue(0, rowbuf, sem)

            def two(i, _):
                g0 = i * 2
                issue(g0 + 1, rowbuf2, sem2)
                waitbuf(rowbuf, sem)
                process(g0, rowbuf)

                @pl.when(g0 + 2 < BLK // 16)
                def _():
                    issue(g0 + 2, rowbuf, sem)

                waitbuf(rowbuf2, sem2)
                process(g0 + 1, rowbuf2)
                return 0
            lax.fori_loop(0, (BLK // 16) // 2, two, 0)
            return 0
        lax.fori_loop(0, nblk, pass2_blk, 0)

        scale = 1.0 / H

        def ep(i, _):
            v = acc[pl.ds(i * 16, 16)] * scale
            col = (i % (F // 16)) * 16
            z = v + bv[pl.ds(col, 16)]
            if H == 4:
                o = jnp.where(z > 0.0, z,
                              BETA * z + (1.0 - BETA) * (jnp.exp(z) - 1.0))
            else:
                o = CC * z
            acc[pl.ds(i * 16, 16)] = o
            return 0
        lax.fori_loop(0, CN * F // 16, ep, 0)
        pltpu.sync_copy(acc, out_hbm.at[pl.ds(base * F, CN * F)])


def _make_sc_edge(H):
    HF = H * F
    mesh = plsc.VectorSubcoreMesh(core_axis_name="c", subcore_axis_name="s")
    return pl.kernel(
        functools.partial(_sc_edge_body, H),
        out_type=jax.ShapeDtypeStruct((NPAD * F,), jnp.float32),
        mesh=mesh,
        compiler_params=pltpu.CompilerParams(needs_layout_passes=False),
        scratch_types=[
            pltpu.VMEM((NPAD * 4,), jnp.float32),
            pltpu.VMEM((CN * 4,), jnp.float32),
            pltpu.VMEM((CN * H,), jnp.float32),
            pltpu.VMEM((CN * F,), jnp.float32),
            pltpu.VMEM((BLK,), jnp.int32),
            pltpu.VMEM((BLK,), jnp.int32),
            pltpu.VMEM((16, HF), jnp.float32),
            pltpu.VMEM((16, HF), jnp.float32),
            pltpu.VMEM((144,), jnp.int32),
            pltpu.VMEM((F,), jnp.float32),
            pltpu.VMEM((32,), jnp.int32),
            pltpu.VMEM((32,), jnp.int32),
            pltpu.VMEM((16 * H + 16,), jnp.float32),
            pltpu.SemaphoreType.DMA,
            pltpu.SemaphoreType.DMA,
        ],
    )


_sc_l1 = _make_sc_edge(H1)
_sc_l2 = _make_sc_edge(1)


def kernel(x, edge_index, edge_weight, W1, a_src1, a_dst1, b1, W2, a_src2,
           a_dst2, b2):
    ei = edge_index.astype(jnp.int32)
    loops = jnp.arange(N, dtype=jnp.int32)
    src = jnp.concatenate([ei[0], loops])
    dst = jnp.concatenate([ei[1], loops])
    order = jnp.argsort(dst)
    src_s = jnp.take(src, order)
    dst_s = jnp.take(dst, order)
    bounds = jnp.arange(NCHUNK + 1, dtype=jnp.int32) * CN
    se = jnp.searchsorted(dst_s, bounds).astype(jnp.int32)
    sev = jnp.concatenate([se[:NCHUNK], se[1:]])
    srcp = jnp.pad(src_s, (0, EPAD - EP))
    dstp = jnp.pad(dst_s, (0, EPAD - EP))
    xp = jnp.pad(x, ((0, NPAD - N), (0, 0)))

    eyeH = jnp.eye(H1, dtype=jnp.float32)
    A1 = jnp.concatenate([
        (a_src1[0][:, :, None] * eyeH[:, None, :]).reshape(H1 * F, H1),
        (a_dst1[0][:, :, None] * eyeH[:, None, :]).reshape(H1 * F, H1),
    ], axis=1)
    A2 = jnp.concatenate([
        jnp.pad(a_src2[0].T, ((0, 0), (0, 3))),
        jnp.pad(a_dst2[0].T, ((0, 0), (0, 3))),
    ], axis=1)

    h1, as1, ad1 = _tc_layer(xp, W1, A1)
    x1f = _sc_l1(h1, as1.reshape(-1), ad1.reshape(-1), srcp, dstp, sev, b1)
    x1 = x1f.reshape(NPAD, F)
    h2, as2, ad2 = _tc_layer(x1, W2, A2)
    outf = _sc_l2(h2, as2.reshape(-1), ad2.reshape(-1), srcp, dstp, sev, b2)
    return outf.reshape(NPAD, F)[:N]


# speedup vs baseline: 1.2445x; 1.2445x over previous
import functools

import jax
import jax.numpy as jnp
from jax import lax
from jax.experimental import pallas as pl
from jax.experimental.pallas import tpu as pltpu
from jax.experimental.pallas import tpu_sc as plsc

N = 10000
NPAD = 10240
D = 256
H1 = 4
F = 256
E = 160000
EP = E + N
BLK = 1024
CN = 160
NCHUNK = NPAD // CN
EPAD = 172032
NEG = 0.2
BETA = 0.5
CC = 1.0


def _tc_body(x_ref, w_ref, a_ref, h_ref, as_ref, ad_ref):
    h = jnp.dot(x_ref[...], w_ref[...], preferred_element_type=jnp.float32)
    h_ref[...] = h
    sd = jnp.dot(h, a_ref[...], preferred_element_type=jnp.float32)
    as_ref[...] = sd[:, :4]
    ad_ref[...] = sd[:, 4:]


def _tc_layer(xp, W, A):
    K, M = W.shape
    R = 2048
    f = pl.pallas_call(
        _tc_body,
        grid=(NPAD // R,),
        in_specs=[
            pl.BlockSpec((R, K), lambda i: (i, 0)),
            pl.BlockSpec((K, M), lambda i: (0, 0)),
            pl.BlockSpec((M, 8), lambda i: (0, 0)),
        ],
        out_specs=[
            pl.BlockSpec((R, M), lambda i: (i, 0)),
            pl.BlockSpec((R, 4), lambda i: (i, 0)),
            pl.BlockSpec((R, 4), lambda i: (i, 0)),
        ],
        out_shape=[
            jax.ShapeDtypeStruct((NPAD, M), jnp.float32),
            jax.ShapeDtypeStruct((NPAD, 4), jnp.float32),
            jax.ShapeDtypeStruct((NPAD, 4), jnp.float32),
        ],
    )
    return f(xp, W, A)


def _ext(sev, pos):
    return sev[pl.ds(pos, 16)][0]


def _sc_edge_body(H, h_hbm, as_hbm, ad_hbm, src_hbm, dst_hbm, se_hbm, b_hbm,
                  out_hbm,
                  asv, adc, sv, acc, srcb, dstb, rb0, rb1, rb2, rb3, sev, bv,
                  dbuf, mbuf, abuf, sm0, sm1, sm2, sm3):
    bufs = [rb0, rb1, rb2, rb3]
    sems = [sm0, sm1, sm2, sm3]
    NB = 4
    NGRP = BLK // 16
    cid = lax.axis_index("c")
    sid = lax.axis_index("s")
    wid = sid * 2 + cid
    io = lax.iota(jnp.int32, 16)
    pltpu.sync_copy(as_hbm, asv)
    pltpu.sync_copy(se_hbm, sev.at[pl.ds(0, 128)])
    pltpu.sync_copy(b_hbm, bv)

    def _edge_w(srcv, doff):
        ws = []
        for h in range(H):
            a_s = plsc.load_gather(asv, [srcv * 4 + h])
            a_d = plsc.load_gather(adc, [doff * 4 + h])
            ev = a_s + a_d
            ev = jnp.where(ev >= 0.0, ev, NEG * ev)
            ws.append(jnp.exp(ev))
        return ws

    for k in range(NCHUNK // 32):
        c = wid * (NCHUNK // 32) + k
        base = c * CN
        start = _ext(sev, c)
        end = _ext(sev, c + NCHUNK)
        e0 = (start // 8) * 8
        nblk = (end - e0 + (BLK - 1)) // BLK
        pltpu.sync_copy(ad_hbm.at[pl.ds(base * 4, CN * 4)], adc)

        zf = jnp.zeros((16,), jnp.float32)

        def zs(i, _):
            sv[pl.ds(i * 16, 16)] = zf
            return 0
        lax.fori_loop(0, CN * H // 16, zs, 0)

        def za(i, _):
            acc[pl.ds(i * 16, 16)] = zf
            return 0
        lax.fori_loop(0, CN * F // 16, za, 0)

        def pass1_blk(b, _):
            eb = e0 + b * BLK
            pltpu.sync_copy(src_hbm.at[pl.ds(eb, BLK)], srcb)
            pltpu.sync_copy(dst_hbm.at[pl.ds(eb, BLK)], dstb)

            def grp(g, _):
                gb = g * 16
                srcv = srcb[pl.ds(gb, 16)]
                dstv = dstb[pl.ds(gb, 16)]
                gidx = eb + gb + io
                m = (gidx >= start) & (gidx < end)
                doff = jnp.clip(dstv - base, 0, CN - 1)
                ws = _edge_w(srcv, doff)
                for h in range(H):
                    plsc.addupdate_scatter(sv, [doff * H + h], ws[h], mask=m)
                return 0
            lax.fori_loop(0, BLK // 16, grp, 0)
            return 0
        lax.fori_loop(0, nblk, pass1_blk, 0)

        def inv(i, _):
            s16 = sv[pl.ds(i * 16, 16)]
            sv[pl.ds(i * 16, 16)] = 1.0 / (s16 + 1e-16)
            return 0
        lax.fori_loop(0, CN * H // 16, inv, 0)

        def pass2_blk(b, _):
            eb = e0 + b * BLK
            pltpu.sync_copy(src_hbm.at[pl.ds(eb, BLK)], srcb)
            pltpu.sync_copy(dst_hbm.at[pl.ds(eb, BLK)], dstb)

            def issue(g, buf, sm):
                srcv = srcb[pl.ds(g * 16, 16)]
                pltpu.async_copy(h_hbm.at[srcv], buf, sm)

            def waitbuf(buf, sm):
                pltpu.make_async_copy(h_hbm.at[pl.ds(0, 16)], buf, sm).wait()

            def process(g, buf):
                gb = g * 16
                srcv = srcb[pl.ds(gb, 16)]
                dstv = dstb[pl.ds(gb, 16)]
                gidx = eb + gb + io
                m = (gidx >= start) & (gidx < end)
                doff = jnp.clip(dstv - base, 0, CN - 1)
                ws = _edge_w(srcv, doff)
                for h in range(H):
                    invv = plsc.load_gather(sv, [doff * H + h])
                    abuf[pl.ds(h * 16, 16)] = ws[h] * invv
                dbuf[pl.ds(0, 16)] = doff
                mbuf[pl.ds(0, 16)] = m.astype(jnp.int32)

                def edge(l, _):
                    ml = mbuf[pl.ds(l, 16)][0] > 0

                    @pl.when(ml)
                    def _():
                        dof = dbuf[pl.ds(l, 16)][0]
                        als = [abuf[pl.ds(h * 16 + l, 16)][0]
                               for h in range(H)]
                        for jb in range(F // 32):
                            vlo = None
                            vhi = None
                            for h in range(H):
                                w = buf[l, pl.ds(h * (F // 2) + jb * 16, 16)]
                                wb = plsc.bitcast(w, jnp.bfloat16)
                                a, bb = plsc.unpack(
                                    wb, format=plsc.PackFormat.INTERLEAVED)
                                if vlo is None:
                                    vlo = als[h] * a
                                    vhi = als[h] * bb
                                else:
                                    vlo = vlo + als[h] * a
                                    vhi = vhi + als[h] * bb
                            o = dof * F + jb * 32
                            plsc.addupdate(acc.at[pl.ds(o, 16)], vlo)
                            plsc.addupdate(acc.at[pl.ds(o + 16, 16)], vhi)
                    return 0
                lax.fori_loop(0, 16, edge, 0)

            for r in range(NB - 1):
                issue(r, bufs[r], sems[r])

            def quad(i, _):
                g = i * NB
                for r in range(NB):
                    nxt = g + r + (NB - 1)
                    pr = (r + NB - 1) % NB

                    @pl.when(nxt < NGRP)
                    def _(nxt=nxt, pr=pr):
                        issue(nxt, bufs[pr], sems[pr])

                    waitbuf(bufs[r], sems[r])
                    process(g + r, bufs[r])
                return 0
            lax.fori_loop(0, NGRP // NB, quad, 0)
            return 0
        lax.fori_loop(0, nblk, pass2_blk, 0)

        scale = 1.0 / H

        def ep(i, _):
            v = acc[pl.ds(i * 16, 16)] * scale
            col = (i % (F // 16)) * 16
            z = v + bv[pl.ds(col, 16)]
            if H == 4:
                o = jnp.where(z > 0.0, z,
                              BETA * z + (1.0 - BETA) * (jnp.exp(z) - 1.0))
            else:
                o = CC * z
            acc[pl.ds(i * 16, 16)] = o
            return 0
        lax.fori_loop(0, CN * F // 16, ep, 0)
        pltpu.sync_copy(acc, out_hbm.at[pl.ds(base * F, CN * F)])


def _make_sc_edge(H):
    HF = H * F
    mesh = plsc.VectorSubcoreMesh(core_axis_name="c", subcore_axis_name="s")
    return pl.kernel(
        functools.partial(_sc_edge_body, H),
        out_type=jax.ShapeDtypeStruct((NPAD * F,), jnp.float32),
        mesh=mesh,
        compiler_params=pltpu.CompilerParams(needs_layout_passes=False),
        scratch_types=[
            pltpu.VMEM((NPAD * 4,), jnp.float32),
            pltpu.VMEM((CN * 4,), jnp.float32),
            pltpu.VMEM((CN * H,), jnp.float32),
            pltpu.VMEM((CN * F,), jnp.float32),
            pltpu.VMEM((BLK,), jnp.int32),
            pltpu.VMEM((BLK,), jnp.int32),
            pltpu.VMEM((16, HF // 2), jnp.int32),
            pltpu.VMEM((16, HF // 2), jnp.int32),
            pltpu.VMEM((16, HF // 2), jnp.int32),
            pltpu.VMEM((16, HF // 2), jnp.int32),
            pltpu.VMEM((144,), jnp.int32),
            pltpu.VMEM((F,), jnp.float32),
            pltpu.VMEM((32,), jnp.int32),
            pltpu.VMEM((32,), jnp.int32),
            pltpu.VMEM((16 * H + 16,), jnp.float32),
            pltpu.SemaphoreType.DMA,
            pltpu.SemaphoreType.DMA,
            pltpu.SemaphoreType.DMA,
            pltpu.SemaphoreType.DMA,
        ],
    )


_sc_l1 = _make_sc_edge(H1)
_sc_l2 = _make_sc_edge(1)


def _pack_rows(h):
    M = h.shape[1]
    hb = h.astype(jnp.bfloat16).reshape(NPAD, M // 32, 2, 16)
    hs = jnp.swapaxes(hb, 2, 3)
    return jax.lax.bitcast_convert_type(hs, jnp.int32).reshape(NPAD, M // 2)


def kernel(x, edge_index, edge_weight, W1, a_src1, a_dst1, b1, W2, a_src2,
           a_dst2, b2):
    ei = edge_index.astype(jnp.int32)
    loops = jnp.arange(N, dtype=jnp.int32)
    src = jnp.concatenate([ei[0], loops])
    dst = jnp.concatenate([ei[1], loops])
    order = jnp.argsort(dst)
    src_s = jnp.take(src, order)
    dst_s = jnp.take(dst, order)
    bounds = jnp.arange(NCHUNK + 1, dtype=jnp.int32) * CN
    se = jnp.searchsorted(dst_s, bounds).astype(jnp.int32)
    sev = jnp.concatenate([se[:NCHUNK], se[1:]])
    srcp = jnp.pad(src_s, (0, EPAD - EP))
    dstp = jnp.pad(dst_s, (0, EPAD - EP))
    xp = jnp.pad(x, ((0, NPAD - N), (0, 0)))

    eyeH = jnp.eye(H1, dtype=jnp.float32)
    A1 = jnp.concatenate([
        (a_src1[0][:, :, None] * eyeH[:, None, :]).reshape(H1 * F, H1),
        (a_dst1[0][:, :, None] * eyeH[:, None, :]).reshape(H1 * F, H1),
    ], axis=1)
    A2 = jnp.concatenate([
        jnp.pad(a_src2[0].T, ((0, 0), (0, 3))),
        jnp.pad(a_dst2[0].T, ((0, 0), (0, 3))),
    ], axis=1)

    h1, as1, ad1 = _tc_layer(xp, W1, A1)
    x1f = _sc_l1(_pack_rows(h1), as1.reshape(-1), ad1.reshape(-1), srcp, dstp,
                 sev, b1)
    x1 = x1f.reshape(NPAD, F)
    h2, as2, ad2 = _tc_layer(x1, W2, A2)
    outf = _sc_l2(_pack_rows(h2), as2.reshape(-1), ad2.reshape(-1), srcp,
                  dstp, sev, b2)
    return outf.reshape(NPAD, F)[:N]
